# baseline (speedup 1.0000x reference)
import jax
import jax.numpy as jnp
from jax import lax
from jax.experimental import pallas as pl
from jax.experimental.pallas import tpu as pltpu
from jax.experimental.pallas import tpu_sc as plsc

N = 1_000_000
NPAD = 1_000_064
C = 32
H = 512
W = 512
HW = H * W
K = 128
NCHUNKS = (N + K - 1) // K
NWORKERS = 32
PLANE_SRC = ((1, 2), (0, 2), (0, 1))


def _sc_body(x_hbm, table_hbm, out_hbm,
             cxa_v, cxb_v, idx_v, w_v, rows_v, out_v,
             csem0, csem1, gsem0, gsem1, osem0, osem1):
    cid = lax.axis_index("c")
    sid = lax.axis_index("s")
    wid = sid * 2 + cid
    nloc = (NCHUNKS + (NWORKERS - 1) - wid) // NWORKERS
    cx_refs = (cxa_v, cxb_v)
    csem = (csem0, csem1)
    gsem = (gsem0, gsem1)
    osem = (osem0, osem1)

    def base_of(i):
        return (wid + i * NWORKERS) * K

    def coord_descs(i, b):
        g = wid + i * NWORKERS
        return [pltpu.make_async_copy(
            x_hbm.at[g, pl.ds(0, 3)], cx_refs[b], csem[b])]

    def gather_desc(i, b, jj):
        return pltpu.make_async_copy(
            table_hbm.at[idx_v.at[b, jj]], rows_v.at[b, jj], gsem[b])

    def out_desc(i, b):
        return pltpu.make_async_copy(
            out_v.at[b],
            out_hbm.at[pl.ds(base_of(i) * C // 128, K * C // 128)], osem[b])

    def phase_a(b):
        for gi in range(K // 16):
            sl = pl.ds(gi * 16, 16)
            cs = tuple(cx_refs[b][col, sl] for col in range(3))
            for q, (wsrc, hsrc) in enumerate(PLANE_SRC):
                gx = cs[wsrc] * 255.5 + 255.5
                gy = cs[hsrc] * 255.5 + 255.5
                x0i = jnp.minimum(jnp.maximum(gx.astype(jnp.int32), 0), W - 2)
                y0i = jnp.minimum(jnp.maximum(gy.astype(jnp.int32), 0), H - 2)
                wx1 = gx - x0i.astype(jnp.float32)
                wy1 = gy - y0i.astype(jnp.float32)
                wx0 = 1.0 - wx1
                wy0 = 1.0 - wy1
                i00 = y0i * W + x0i + q * HW
                bq = 4 * q
                idx_v[b, bq + 0, sl] = i00
                idx_v[b, bq + 1, sl] = i00 + 1
                idx_v[b, bq + 2, sl] = i00 + W
                idx_v[b, bq + 3, sl] = i00 + W + 1
                w_v[b, bq + 0, sl] = wy0 * wx0
                w_v[b, bq + 1, sl] = wy0 * wx1
                w_v[b, bq + 2, sl] = wy1 * wx0
                w_v[b, bq + 3, sl] = wy1 * wx1

    def combine(b):
        def grp_body(gi, carry):
            sl = pl.ds(gi * 16, 16)
            wv = [w_v[b, jj, sl] for jj in range(12)]
            for pp in range(16):
                p = gi * 16 + pp
                acc0 = jnp.zeros((16,), jnp.float32)
                acc1 = jnp.zeros((16,), jnp.float32)
                for jj in range(12):
                    wgt = wv[jj][pp]
                    acc0 = acc0 + rows_v[b, jj, p, pl.ds(0, 16)] * wgt
                    acc1 = acc1 + rows_v[b, jj, p, pl.ds(16, 16)] * wgt
                orow = gi * 4 + pp // 4
                ocol = (pp % 4) * C
                out_v[b, orow, pl.ds(ocol, 16)] = acc0
                out_v[b, orow, pl.ds(ocol + 16, 16)] = acc1
            return carry

        lax.fori_loop(0, K // 16, grp_body, 0)

    for cd in coord_descs(0, 0):
        cd.start()
    for cd in coord_descs(0, 0):
        cd.wait()
    phase_a(0)
    for jj in range(12):
        gather_desc(0, 0, jj).start()
    for cd in coord_descs(1, 1):
        cd.start()

    npairs = (nloc + 1) // 2

    def pair_body(ip, carry):
        for b in (0, 1):
            i = ip * 2 + b

            @pl.when(i < nloc)
            def _():
                nb = 1 - b
                for jj in range(12):
                    gather_desc(i, b, jj).wait()

                @pl.when(i + 1 < nloc)
                def _():
                    for cd in coord_descs(i + 1, nb):
                        cd.wait()
                    phase_a(nb)
                    for jj in range(12):
                        gather_desc(i + 1, nb, jj).start()

                    @pl.when(i + 2 < nloc)
                    def _():
                        for cd in coord_descs(i + 2, b):
                            cd.start()

                @pl.when(i >= 2)
                def _():
                    out_desc(i - 2, b).wait()

                combine(b)
                out_desc(i, b).start()

        return carry

    lax.fori_loop(0, npairs, pair_body, 0)

    for b in (0, 1):
        i_b = nloc - 1 - ((nloc - 1 - b) % 2)
        out_desc(i_b, b).wait()


def kernel(x, planes):
    table = jnp.transpose(planes, (0, 2, 3, 1)).reshape(3 * HW, C)
    xt = jnp.pad(x.T, ((0, 5), (0, NPAD - N)))
    xt = xt.reshape(8, NPAD // 128, 128).transpose(1, 0, 2)
    mesh = plsc.VectorSubcoreMesh(core_axis_name="c", subcore_axis_name="s")
    fn = pl.kernel(
        _sc_body,
        mesh=mesh,
        compiler_params=pltpu.CompilerParams(use_tc_tiling_on_sc=False),
        out_type=jax.ShapeDtypeStruct((NPAD * C // 128, 128), jnp.float32),
        scratch_types=[
            pltpu.VMEM((3, K), jnp.float32),
            pltpu.VMEM((3, K), jnp.float32),
            pltpu.VMEM((2, 12, K), jnp.int32),
            pltpu.VMEM((2, 12, K), jnp.float32),
            pltpu.VMEM((2, 12, K, C), jnp.float32),
            pltpu.VMEM((2, K * C // 128, 128), jnp.float32),
            pltpu.SemaphoreType.DMA,
            pltpu.SemaphoreType.DMA,
            pltpu.SemaphoreType.DMA,
            pltpu.SemaphoreType.DMA,
            pltpu.SemaphoreType.DMA,
            pltpu.SemaphoreType.DMA,
        ],
    )
    return fn(xt, table).reshape(NPAD, C)[:N]


# speedup vs baseline: 1.0401x; 1.0401x over previous
import jax
import jax.numpy as jnp
from jax import lax
from jax.experimental import pallas as pl
from jax.experimental.pallas import tpu as pltpu
from jax.experimental.pallas import tpu_sc as plsc

N = 1_000_000
C = 32
H = 512
W = 512
HW = H * W
K = 128
NCHUNKS = (N + K - 1) // K
NWORKERS = 32
PLANE_SRC = ((1, 2), (0, 2), (0, 1))


def _sc_body(x_hbm, table_hbm, out_hbm,
             cxa_v, cxb_v, idx_v, w_v, rows_v, out_v,
             csem0, csem1, gsem0, gsem1, osem0, osem1):
    cid = lax.axis_index("c")
    sid = lax.axis_index("s")
    wid = sid * 2 + cid
    nloc = (NCHUNKS + (NWORKERS - 1) - wid) // NWORKERS
    cx_refs = (cxa_v, cxb_v)
    csem = (csem0, csem1)
    gsem = (gsem0, gsem1)
    osem = (osem0, osem1)

    def base_of(i):
        return jnp.minimum((wid + i * NWORKERS) * K, N - K)

    def coord_descs(i, b):
        base = base_of(i)
        return [
            pltpu.make_async_copy(
                x_hbm.at[pl.ds(col * N + base, K)], cx_refs[b].at[col],
                csem[b])
            for col in range(3)
        ]

    def gather_desc(i, b, jj):
        return pltpu.make_async_copy(
            table_hbm.at[idx_v.at[b, jj]], rows_v.at[b, jj], gsem[b])

    def out_desc(i, b):
        return pltpu.make_async_copy(
            out_v.at[b],
            out_hbm.at[pl.ds(base_of(i) * C // 128, K * C // 128)], osem[b])

    def phase_a(b):
        for gi in range(K // 16):
            sl = pl.ds(gi * 16, 16)
            cs = tuple(cx_refs[b][col, sl] for col in range(3))
            for q, (wsrc, hsrc) in enumerate(PLANE_SRC):
                gx = cs[wsrc] * 255.5 + 255.5
                gy = cs[hsrc] * 255.5 + 255.5
                x0i = jnp.minimum(jnp.maximum(gx.astype(jnp.int32), 0), W - 2)
                y0i = jnp.minimum(jnp.maximum(gy.astype(jnp.int32), 0), H - 2)
                wx1 = gx - x0i.astype(jnp.float32)
                wy1 = gy - y0i.astype(jnp.float32)
                wx0 = 1.0 - wx1
                wy0 = 1.0 - wy1
                i00 = y0i * W + x0i + q * HW
                bq = 4 * q
                idx_v[b, bq + 0, sl] = i00
                idx_v[b, bq + 1, sl] = i00 + 1
                idx_v[b, bq + 2, sl] = i00 + W
                idx_v[b, bq + 3, sl] = i00 + W + 1
                w_v[b, bq + 0, sl] = wy0 * wx0
                w_v[b, bq + 1, sl] = wy0 * wx1
                w_v[b, bq + 2, sl] = wy1 * wx0
                w_v[b, bq + 3, sl] = wy1 * wx1

    def combine(b):
        def grp_body(gi, carry):
            sl = pl.ds(gi * 16, 16)
            wv = [w_v[b, jj, sl] for jj in range(12)]
            for pp in range(16):
                p = gi * 16 + pp
                acc0 = jnp.zeros((16,), jnp.float32)
                acc1 = jnp.zeros((16,), jnp.float32)
                for jj in range(12):
                    wgt = wv[jj][pp]
                    acc0 = acc0 + rows_v[b, jj, p, pl.ds(0, 16)] * wgt
                    acc1 = acc1 + rows_v[b, jj, p, pl.ds(16, 16)] * wgt
                orow = gi * 4 + pp // 4
                ocol = (pp % 4) * C
                out_v[b, orow, pl.ds(ocol, 16)] = acc0
                out_v[b, orow, pl.ds(ocol + 16, 16)] = acc1
            return carry

        lax.fori_loop(0, K // 16, grp_body, 0)

    for cd in coord_descs(0, 0):
        cd.start()
    for cd in coord_descs(0, 0):
        cd.wait()
    phase_a(0)
    for jj in range(12):
        gather_desc(0, 0, jj).start()
    for cd in coord_descs(1, 1):
        cd.start()

    npairs = (nloc + 1) // 2

    def pair_body(ip, carry):
        for b in (0, 1):
            i = ip * 2 + b

            @pl.when(i < nloc)
            def _():
                nb = 1 - b
                for jj in range(12):
                    gather_desc(i, b, jj).wait()

                @pl.when(i + 1 < nloc)
                def _():
                    for cd in coord_descs(i + 1, nb):
                        cd.wait()
                    phase_a(nb)
                    for jj in range(12):
                        gather_desc(i + 1, nb, jj).start()

                    @pl.when(i + 2 < nloc)
                    def _():
                        for cd in coord_descs(i + 2, b):
                            cd.start()

                @pl.when(i >= 2)
                def _():
                    out_desc(i - 2, b).wait()

                combine(b)
                out_desc(i, b).start()

        return carry

    lax.fori_loop(0, npairs, pair_body, 0)

    for b in (0, 1):
        i_b = nloc - 1 - ((nloc - 1 - b) % 2)
        out_desc(i_b, b).wait()


def kernel(x, planes):
    table = jnp.transpose(planes, (0, 2, 3, 1)).reshape(3 * HW, C)
    xt = x.T.reshape(3 * N)
    mesh = plsc.VectorSubcoreMesh(core_axis_name="c", subcore_axis_name="s")
    fn = pl.kernel(
        _sc_body,
        mesh=mesh,
        compiler_params=pltpu.CompilerParams(use_tc_tiling_on_sc=False),
        out_type=jax.ShapeDtypeStruct((N * C // 128, 128), jnp.float32),
        scratch_types=[
            pltpu.VMEM((3, K), jnp.float32),
            pltpu.VMEM((3, K), jnp.float32),
            pltpu.VMEM((2, 12, K), jnp.int32),
            pltpu.VMEM((2, 12, K), jnp.float32),
            pltpu.VMEM((2, 12, K, C), jnp.float32),
            pltpu.VMEM((2, K * C // 128, 128), jnp.float32),
            pltpu.SemaphoreType.DMA,
            pltpu.SemaphoreType.DMA,
            pltpu.SemaphoreType.DMA,
            pltpu.SemaphoreType.DMA,
            pltpu.SemaphoreType.DMA,
            pltpu.SemaphoreType.DMA,
        ],
    )
    return fn(xt, table).reshape(N, C)
